# baseline (speedup 1.0000x reference)
import math

import jax
import jax.numpy as jnp
from jax import lax
from jax.experimental import pallas as pl
from jax.experimental.pallas import tpu as pltpu
from jax.experimental.pallas import tpu_sc as plsc

VOCAB = 100000
D = 64
S = 200
B = 1024
NW = 32
SBLK = 8
BBLK = 128
NUNITS = (S // SBLK) * (B // BBLK)
UMAX = -(-NUNITS // NW)
CPAD = D + 1
TPAD = 2 * D + 2
SCALE = math.sqrt(D)


def _positional_encoding():
    pos = jnp.arange(S, dtype=jnp.float32)[:, None]
    div = jnp.exp(jnp.arange(0, D, 2, dtype=jnp.float32) * (-math.log(10000.0) / D))
    ang = pos * div[None, :]
    pe = jnp.zeros((S, D), dtype=jnp.float32)
    pe = pe.at[:, 0::2].set(jnp.sin(ang))
    pe = pe.at[:, 1::2].set(jnp.cos(ang))
    return pe


def _emb_kernel(table, ids_t, tts_t, pe_in, ttab_in, out,
                comb, pebuf, tbuf, idxv, ttv, ttfbuf,
                rows0, rows1, tile0, tile1,
                gsem0, gsem1, wsem0, wsem1):
    nc = 2
    wid = lax.axis_index("s") * nc + lax.axis_index("c")
    rows = (rows0, rows1)
    tiles = (tile0, tile1)
    gsem = (gsem0, gsem1)
    wsem = (wsem0, wsem1)
    iota16 = lax.iota(jnp.int32, 16)

    pltpu.sync_copy(pe_in, pebuf)
    pltpu.sync_copy(ttab_in, tbuf)
    for jq in range(D // 16):
        sl = pl.ds(16 * jq, 16)
        jvec0 = iota16 + 16 * jq
        t0 = tbuf[0, sl]
        t1 = tbuf[1, sl]

        def _add_body(s, carry):
            a, b2 = carry
            pv = pebuf[s, sl]
            adr = jvec0 + s * CPAD
            plsc.store_scatter(comb, [adr], pv + a)
            plsc.store_scatter(comb, [adr + S * CPAD], pv + b2)
            return carry

        lax.fori_loop(0, S, _add_body, (t0, t1))

    def _unit_body(u, carry):
        unit = wid + NW * u

        @pl.when(unit < NUNITS)
        def _run():
            s_base = pl.multiple_of((unit // (B // BBLK)) * SBLK, SBLK)
            b0 = pl.multiple_of((unit % (B // BBLK)) * BBLK, BBLK)
            bt = unit % (B // BBLK)

            pltpu.sync_copy(ids_t.at[pl.ds(s_base, SBLK), pl.ds(b0, BBLK)],
                            idxv)
            pltpu.sync_copy(tts_t.at[pl.ds(s_base, SBLK), pl.ds(b0, BBLK)],
                            ttv)

            def fire(si):
                pltpu.async_copy(table.at[idxv.at[si]], rows[si % 2],
                                 gsem[si % 2])

            def wait_g(si):
                pltpu.make_async_copy(table.at[idxv.at[si]], rows[si % 2],
                                      gsem[si % 2]).wait()

            def compute(si):
                s = s_base + si
                rv, tl = rows[si % 2], tiles[si % 2]
                for bg in range(BBLK // 16):
                    bsl = pl.ds(bg * 16, 16)
                    ttfbuf[bsl] = ttv[si, bsl].astype(jnp.float32)
                cbase = s * CPAD

                def _jd_body(jq, c2):
                    for jr in range(2):
                        jvec = (jq * 2 + jr + iota16) & (D - 1)
                        cadr = cbase + jvec
                        c0 = plsc.load_gather(comb, [cadr])
                        c1 = plsc.load_gather(comb, [cadr + S * CPAD])
                        dvec = c1 - c0
                        for bg in range(BBLK // 16):
                            bvec = iota16 + bg * 16
                            tvec = ttfbuf[pl.ds(bg * 16, 16)]
                            evec = plsc.load_gather(rv, [bvec, jvec])
                            res = evec * SCALE + (c0 + tvec * dvec)
                            plsc.store_scatter(tl, [jvec, bvec], res)
                    return c2

                lax.fori_loop(0, D // 2, _jd_body, 0)

            def writeback(si):
                s = s_base + si
                for jt in range(8):
                    pltpu.async_copy(
                        tiles[si % 2].at[pl.ds(jt * 8, 8), pl.ds(0, BBLK)],
                        out.at[s, jt, bt],
                        wsem[si % 2])

            def wait_wb(si):
                s = s_base + si
                for jt in range(8):
                    pltpu.make_async_copy(
                        tiles[si % 2].at[pl.ds(jt * 8, 8), pl.ds(0, BBLK)],
                        out.at[s, jt, bt],
                        wsem[si % 2]).wait()

            @pl.when(u > 0)
            def _drain():
                for p in range(2):
                    for jt in range(8):
                        pltpu.make_async_copy(
                            tiles[p].at[pl.ds(jt * 8, 8), pl.ds(0, BBLK)],
                            out.at[0, jt, 0],
                            wsem[p]).wait()

            fire(0)
            for si in range(SBLK):
                if si + 1 < SBLK:
                    fire(si + 1)
                wait_g(si)
                if si >= 2:
                    wait_wb(si - 2)
                compute(si)
                writeback(si)

        return carry

    lax.fori_loop(0, UMAX, _unit_body, 0)

    for p in range(2):
        for jt in range(8):
            pltpu.make_async_copy(
                tiles[p].at[pl.ds(jt * 8, 8), pl.ds(0, BBLK)],
                out.at[0, jt, 0],
                wsem[p]).wait()


def kernel(input_ids, token_type_ids, embedding_table, token_type_table):
    ids_t = input_ids.astype(jnp.int32).T
    tts_t = token_type_ids.astype(jnp.int32).T
    pe = _positional_encoding()

    mesh = plsc.VectorSubcoreMesh(core_axis_name="c", subcore_axis_name="s")
    run = pl.kernel(
        _emb_kernel,
        mesh=mesh,
        out_type=jax.ShapeDtypeStruct((S, 8, B // BBLK, 8, BBLK), jnp.float32),
        compiler_params=pltpu.CompilerParams(use_tc_tiling_on_sc=False,
                                             needs_layout_passes=False),
        scratch_types=[
            pltpu.VMEM((2 * S * CPAD,), jnp.float32),
            pltpu.VMEM((S, D), jnp.float32),
            pltpu.VMEM((2, D), jnp.float32),
            pltpu.VMEM((SBLK, BBLK), jnp.int32),
            pltpu.VMEM((SBLK, BBLK), jnp.int32),
            pltpu.VMEM((BBLK,), jnp.float32),
            pltpu.VMEM((BBLK, D), jnp.float32),
            pltpu.VMEM((BBLK, D), jnp.float32),
            pltpu.VMEM((D, TPAD), jnp.float32),
            pltpu.VMEM((D, TPAD), jnp.float32),
            pltpu.SemaphoreType.DMA,
            pltpu.SemaphoreType.DMA,
            pltpu.SemaphoreType.DMA,
            pltpu.SemaphoreType.DMA,
        ],
    )
    out = run(embedding_table, ids_t, tts_t, pe, token_type_table)
    x = jnp.transpose(out, (2, 4, 0, 1, 3))
    return x.reshape(B, S, D)


# speedup vs baseline: 1.0860x; 1.0860x over previous
import math

import jax
import jax.numpy as jnp
from jax import lax
from jax.experimental import pallas as pl
from jax.experimental.pallas import tpu as pltpu
from jax.experimental.pallas import tpu_sc as plsc

VOCAB = 100000
D = 64
S = 200
B = 1024
NW = 32
SBLK = 8
BBLK = 128
NUNITS = (S // SBLK) * (B // BBLK)
UMAX = -(-NUNITS // NW)
CPAD = D + 1
TPAD = 2 * D + 2
SCALE = math.sqrt(D)


def _positional_encoding():
    pos = jnp.arange(S, dtype=jnp.float32)[:, None]
    div = jnp.exp(jnp.arange(0, D, 2, dtype=jnp.float32) * (-math.log(10000.0) / D))
    ang = pos * div[None, :]
    pe = jnp.zeros((S, D), dtype=jnp.float32)
    pe = pe.at[:, 0::2].set(jnp.sin(ang))
    pe = pe.at[:, 1::2].set(jnp.cos(ang))
    return pe


def _emb_kernel(table, ids_t, tts_t, pe_in, ttab_in, out,
                comb, pebuf, tbuf, idxv, ttv, rows0, rows1, tile0, tile1,
                gsem0, gsem1, wsem0, wsem1):
    nc = 2
    wid = lax.axis_index("s") * nc + lax.axis_index("c")
    rows = (rows0, rows1)
    tiles = (tile0, tile1)
    gsem = (gsem0, gsem1)
    wsem = (wsem0, wsem1)
    iota16 = lax.iota(jnp.int32, 16)

    pltpu.sync_copy(pe_in, pebuf)
    pltpu.sync_copy(ttab_in, tbuf)
    for jq in range(D // 16):
        sl = pl.ds(16 * jq, 16)
        jvec0 = iota16 + 16 * jq
        t0 = tbuf[0, sl]
        t1 = tbuf[1, sl]

        def _add_body(s, carry):
            a, b2 = carry
            pv = pebuf[s, sl]
            adr = jvec0 + s * CPAD
            plsc.store_scatter(comb, [adr], pv + a)
            plsc.store_scatter(comb, [adr + S * CPAD], pv + b2)
            return carry

        lax.fori_loop(0, S, _add_body, (t0, t1))

    def _unit_body(u, carry):
        unit = wid + NW * u

        @pl.when(unit < NUNITS)
        def _run():
            s_base = pl.multiple_of((unit // (B // BBLK)) * SBLK, SBLK)
            b0 = pl.multiple_of((unit % (B // BBLK)) * BBLK, BBLK)
            bt = unit % (B // BBLK)

            pltpu.sync_copy(ids_t.at[pl.ds(s_base, SBLK), pl.ds(b0, BBLK)],
                            idxv)
            pltpu.sync_copy(tts_t.at[pl.ds(s_base, SBLK), pl.ds(b0, BBLK)],
                            ttv)

            def fire(si):
                pltpu.async_copy(table.at[idxv.at[si]], rows[si % 2],
                                 gsem[si % 2])

            def wait_g(si):
                pltpu.make_async_copy(table.at[idxv.at[si]], rows[si % 2],
                                      gsem[si % 2]).wait()

            def compute(si):
                s = s_base + si
                rv, tl = rows[si % 2], tiles[si % 2]

                def _bg_body(bg, c2):
                    ttvec = ttv[si, pl.ds(bg * 16, 16)]
                    kvec = (ttvec * S + s) * CPAD
                    bvec = iota16 + bg * 16

                    def _jd_body(jq, c3):
                        for jr in range(4):
                            jvec = (jq * 4 + jr + iota16) & (D - 1)
                            evec = plsc.load_gather(rv, [bvec, jvec])
                            cvec = plsc.load_gather(comb, [kvec + jvec])
                            plsc.store_scatter(tl, [jvec, bvec],
                                               evec * SCALE + cvec)
                        return c3

                    lax.fori_loop(0, D // 4, _jd_body, 0)
                    return c2

                lax.fori_loop(0, BBLK // 16, _bg_body, 0)

            def writeback(si):
                s = s_base + si
                for jt in range(8):
                    pltpu.async_copy(
                        tiles[si % 2].at[pl.ds(jt * 8, 8), pl.ds(0, BBLK)],
                        out.at[s, jt, bt],
                        wsem[si % 2])

            def wait_wb(si):
                s = s_base + si
                for jt in range(8):
                    pltpu.make_async_copy(
                        tiles[si % 2].at[pl.ds(jt * 8, 8), pl.ds(0, BBLK)],
                        out.at[s, jt, bt],
                        wsem[si % 2]).wait()

            @pl.when(u > 0)
            def _drain():
                for p in range(2):
                    for jt in range(8):
                        pltpu.make_async_copy(
                            tiles[p].at[pl.ds(jt * 8, 8), pl.ds(0, BBLK)],
                            out.at[0, jt, 0],
                            wsem[p]).wait()

            fire(0)
            for si in range(SBLK):
                if si + 1 < SBLK:
                    fire(si + 1)
                wait_g(si)
                if si >= 2:
                    wait_wb(si - 2)
                compute(si)
                writeback(si)

        return carry

    lax.fori_loop(0, UMAX, _unit_body, 0)

    for p in range(2):
        for jt in range(8):
            pltpu.make_async_copy(
                tiles[p].at[pl.ds(jt * 8, 8), pl.ds(0, BBLK)],
                out.at[0, jt, 0],
                wsem[p]).wait()


def kernel(input_ids, token_type_ids, embedding_table, token_type_table):
    ids_t = input_ids.astype(jnp.int32).T
    tts_t = token_type_ids.astype(jnp.int32).T
    pe = _positional_encoding()

    mesh = plsc.VectorSubcoreMesh(core_axis_name="c", subcore_axis_name="s")
    run = pl.kernel(
        _emb_kernel,
        mesh=mesh,
        out_type=jax.ShapeDtypeStruct((S, 8, B // BBLK, 8, BBLK), jnp.float32),
        compiler_params=pltpu.CompilerParams(use_tc_tiling_on_sc=False,
                                             needs_layout_passes=False),
        scratch_types=[
            pltpu.VMEM((2 * S * CPAD,), jnp.float32),
            pltpu.VMEM((S, D), jnp.float32),
            pltpu.VMEM((2, D), jnp.float32),
            pltpu.VMEM((SBLK, BBLK), jnp.int32),
            pltpu.VMEM((SBLK, BBLK), jnp.int32),
            pltpu.VMEM((BBLK, D), jnp.float32),
            pltpu.VMEM((BBLK, D), jnp.float32),
            pltpu.VMEM((D, TPAD), jnp.float32),
            pltpu.VMEM((D, TPAD), jnp.float32),
            pltpu.SemaphoreType.DMA,
            pltpu.SemaphoreType.DMA,
            pltpu.SemaphoreType.DMA,
            pltpu.SemaphoreType.DMA,
        ],
    )
    out = run(embedding_table, ids_t, tts_t, pe, token_type_table)
    x = jnp.transpose(out, (2, 4, 0, 1, 3))
    return x.reshape(B, S, D)
